# baseline (speedup 1.0000x reference)
import functools

import jax
import jax.numpy as jnp
from jax import lax
from jax.experimental import pallas as pl
from jax.experimental.pallas import tpu as pltpu

POOL = 64
K = 8
B = 64
SEQ = 197
D = 768

TCH = 24
NFULL = SEQ // TCH
TAIL = SEQ - NFULL * TCH


def _l2norm_rows(x):
    sq = jnp.sum(x * x, axis=1, keepdims=True)
    return x * lax.rsqrt(jnp.maximum(sq, 1e-12))


def _body(x_hbm, key_ref, sim_ref, bkn_ref, rs_ref, idx_ref,
          xmax_ref, sems, *bufs):
    copies = [
        pltpu.make_async_copy(x_hbm.at[:, pl.ds(j * TCH, TCH), :],
                              bufs[j], sems.at[j])
        for j in range(NFULL)
    ]
    copies.append(
        pltpu.make_async_copy(x_hbm.at[:, pl.ds(NFULL * TCH, TAIL), :],
                              bufs[NFULL], sems.at[NFULL]))
    for c in copies:
        c.start()
    copies[0].wait()
    xmax_ref[...] = jnp.max(bufs[0][...], axis=1)
    for j in range(1, NFULL + 1):
        copies[j].wait()
        xmax_ref[...] = jnp.maximum(xmax_ref[...],
                                    jnp.max(bufs[j][...], axis=1))

    x_max = xmax_ref[...]
    k_norm = _l2norm_rows(key_ref[...])
    x_norm = _l2norm_rows(x_max)
    sim = lax.dot_general(
        x_norm, k_norm, (((1,), (1,)), ((), ())),
        preferred_element_type=jnp.float32)
    sim_ref[...] = sim

    iota = lax.broadcasted_iota(jnp.int32, (B, POOL), 1)
    work = sim
    total = jnp.float32(0.0)
    for kk in range(K):
        m = jnp.max(work, axis=1, keepdims=True)
        amax = jnp.min(jnp.where(work == m, iota, POOL),
                       axis=1, keepdims=True)
        idx_ref[:, kk:kk + 1] = amax
        onehot = (iota == amax).astype(jnp.float32)
        row = jnp.dot(onehot, k_norm, precision=lax.Precision.HIGHEST,
                      preferred_element_type=jnp.float32)
        bkn_ref[:, kk, :] = row
        total = total + jnp.sum(row * x_norm)
        work = jnp.where(iota == amax, -jnp.inf, work)
    rs_ref[...] = jnp.broadcast_to(total / jnp.float32(B), (1, 1))


@functools.partial(jax.jit, static_argnames=("interpret",))
def kernel(x_embed, prompt_key, interpret=False):
    sim, bkn, rs, idx = pl.pallas_call(
        _body,
        in_specs=[
            pl.BlockSpec(memory_space=pl.ANY),
            pl.BlockSpec((POOL, D), lambda: (0, 0)),
        ],
        out_specs=[
            pl.BlockSpec((B, POOL), lambda: (0, 0)),
            pl.BlockSpec((B, K, D), lambda: (0, 0, 0)),
            pl.BlockSpec((1, 1), lambda: (0, 0)),
            pl.BlockSpec((B, K), lambda: (0, 0)),
        ],
        out_shape=[
            jax.ShapeDtypeStruct((B, POOL), jnp.float32),
            jax.ShapeDtypeStruct((B, K, D), jnp.float32),
            jax.ShapeDtypeStruct((1, 1), jnp.float32),
            jax.ShapeDtypeStruct((B, K), jnp.int32),
        ],
        scratch_shapes=(
            [pltpu.VMEM((B, D), jnp.float32),
             pltpu.SemaphoreType.DMA((NFULL + 1,))]
            + [pltpu.VMEM((B, TCH, D), jnp.float32) for _ in range(NFULL)]
            + [pltpu.VMEM((B, TAIL, D), jnp.float32)]
        ),
        interpret=interpret,
    )(x_embed, prompt_key)
    return sim, bkn, rs[0, 0], idx


# speedup vs baseline: 2.9078x; 2.9078x over previous
import functools

import jax
import jax.numpy as jnp
from jax import lax
from jax.experimental import pallas as pl
from jax.experimental.pallas import tpu as pltpu

POOL = 64
K = 8
B = 64
SEQ = 197
D = 768

TCH = 25
NCH = 8
TAILV = SEQ - (NCH - 1) * TCH


def _l2norm_rows(x):
    sq = jnp.sum(x * x, axis=1, keepdims=True)
    return x * lax.rsqrt(jnp.maximum(sq, 1e-12))


def _body(x_ref, key_ref, sim_ref, bkn_ref, rs_ref, idx_ref, xmax_ref):
    i = pl.program_id(0)

    def fold(m):
        @pl.when(i == 0)
        def _():
            xmax_ref[...] = m

        @pl.when(i > 0)
        def _():
            xmax_ref[...] = jnp.maximum(xmax_ref[...], m)

    @pl.when(i < NCH - 1)
    def _():
        fold(jnp.max(x_ref[...], axis=0))

    @pl.when(i == NCH - 1)
    def _():
        fold(jnp.max(x_ref[0:TAILV], axis=0))

    @pl.when(i == NCH - 1)
    def _routing():
        x_max = xmax_ref[...]
        k_norm = _l2norm_rows(key_ref[...])
        x_norm = _l2norm_rows(x_max)
        sim = lax.dot_general(
            x_norm, k_norm, (((1,), (1,)), ((), ())),
            preferred_element_type=jnp.float32)
        sim_ref[...] = sim

        iota = lax.broadcasted_iota(jnp.int32, (B, POOL), 1)
        work = sim
        total = jnp.float32(0.0)
        for kk in range(K):
            m = jnp.max(work, axis=1, keepdims=True)
            amax = jnp.min(jnp.where(work == m, iota, POOL),
                           axis=1, keepdims=True)
            idx_ref[:, kk:kk + 1] = amax
            onehot = (iota == amax).astype(jnp.float32)
            row = jnp.dot(onehot, k_norm,
                          precision=lax.Precision.HIGHEST,
                          preferred_element_type=jnp.float32)
            bkn_ref[:, kk, :] = row
            total = total + jnp.sum(row * x_norm)
            work = jnp.where(iota == amax, -jnp.inf, work)
        rs_ref[...] = jnp.broadcast_to(total / jnp.float32(B), (1, 1))


@functools.partial(jax.jit, static_argnames=("interpret",))
def kernel(x_embed, prompt_key, interpret=False):
    x_t = jnp.transpose(x_embed, (1, 0, 2))
    sim, bkn, rs, idx = pl.pallas_call(
        _body,
        grid=(NCH,),
        in_specs=[
            pl.BlockSpec((TCH, B, D), lambda i: (i, 0, 0)),
            pl.BlockSpec((POOL, D), lambda i: (0, 0)),
        ],
        out_specs=[
            pl.BlockSpec((B, POOL), lambda i: (0, 0)),
            pl.BlockSpec((B, K, D), lambda i: (0, 0, 0)),
            pl.BlockSpec((1, 1), lambda i: (0, 0)),
            pl.BlockSpec((B, K), lambda i: (0, 0)),
        ],
        out_shape=[
            jax.ShapeDtypeStruct((B, POOL), jnp.float32),
            jax.ShapeDtypeStruct((B, K, D), jnp.float32),
            jax.ShapeDtypeStruct((1, 1), jnp.float32),
            jax.ShapeDtypeStruct((B, K), jnp.int32),
        ],
        scratch_shapes=[pltpu.VMEM((B, D), jnp.float32)],
        compiler_params=pltpu.CompilerParams(
            dimension_semantics=("arbitrary",)),
        interpret=interpret,
    )(x_t, prompt_key)
    return sim, bkn, rs[0, 0], idx


# speedup vs baseline: 3.2811x; 1.1284x over previous
import functools

import jax
import jax.numpy as jnp
from jax import lax
from jax.experimental import pallas as pl
from jax.experimental.pallas import tpu as pltpu

POOL = 64
K = 8
B = 64
SEQ = 197
D = 768

TCH = 25
NCH = 8
TAILV = SEQ - (NCH - 1) * TCH


def _l2norm_rows(x):
    sq = jnp.sum(x * x, axis=1, keepdims=True)
    return x * lax.rsqrt(jnp.maximum(sq, 1e-12))


def _body(x_ref, key_ref, sim_ref, bkn_ref, rs_ref, idx_ref, xmax_ref):
    i = pl.program_id(0)

    def fold(m):
        @pl.when(i == 0)
        def _():
            xmax_ref[...] = m

        @pl.when(i > 0)
        def _():
            xmax_ref[...] = jnp.maximum(xmax_ref[...], m)

    @pl.when(i < NCH - 1)
    def _():
        fold(jnp.max(x_ref[...], axis=0))

    @pl.when(i == NCH - 1)
    def _():
        fold(jnp.max(x_ref[0:TAILV], axis=0))

    @pl.when(i == NCH - 1)
    def _routing():
        x_max = xmax_ref[...]
        k_norm = _l2norm_rows(key_ref[...])
        x_norm = _l2norm_rows(x_max)
        sim = lax.dot_general(
            x_norm, k_norm, (((1,), (1,)), ((), ())),
            preferred_element_type=jnp.float32)
        sim_ref[...] = sim

        iota = lax.broadcasted_iota(jnp.int32, (POOL, B), 0)
        work = jnp.transpose(sim)
        total = jnp.float32(0.0)
        for kk in range(K):
            m = jnp.max(work, axis=0, keepdims=True)
            amax = jnp.min(jnp.where(work == m, iota, POOL),
                           axis=0, keepdims=True)
            idx_ref[kk:kk + 1, :] = amax
            onehot = (iota == amax).astype(jnp.float32)
            row = lax.dot_general(
                onehot, k_norm, (((0,), (0,)), ((), ())),
                precision=lax.Precision.HIGHEST,
                preferred_element_type=jnp.float32)
            bkn_ref[:, kk, :] = row
            total = total + jnp.sum(row * x_norm)
            work = jnp.where(iota == amax, -jnp.inf, work)
        rs_ref[...] = jnp.broadcast_to(total / jnp.float32(B), (1, 1))


@functools.partial(jax.jit, static_argnames=("interpret",))
def kernel(x_embed, prompt_key, interpret=False):
    x_t = jnp.transpose(x_embed, (1, 0, 2))
    sim, bkn, rs, idx_t = pl.pallas_call(
        _body,
        grid=(NCH,),
        in_specs=[
            pl.BlockSpec((TCH, B, D), lambda i: (i, 0, 0)),
            pl.BlockSpec((POOL, D), lambda i: (0, 0)),
        ],
        out_specs=[
            pl.BlockSpec((B, POOL), lambda i: (0, 0)),
            pl.BlockSpec((B, K, D), lambda i: (0, 0, 0)),
            pl.BlockSpec((1, 1), lambda i: (0, 0)),
            pl.BlockSpec((K, B), lambda i: (0, 0)),
        ],
        out_shape=[
            jax.ShapeDtypeStruct((B, POOL), jnp.float32),
            jax.ShapeDtypeStruct((B, K, D), jnp.float32),
            jax.ShapeDtypeStruct((1, 1), jnp.float32),
            jax.ShapeDtypeStruct((K, B), jnp.int32),
        ],
        scratch_shapes=[pltpu.VMEM((B, D), jnp.float32)],
        compiler_params=pltpu.CompilerParams(
            dimension_semantics=("arbitrary",)),
        interpret=interpret,
    )(x_t, prompt_key)
    return sim, bkn, rs[0, 0], jnp.transpose(idx_t)
